# baseline (speedup 1.0000x reference)
import functools

import jax
import jax.numpy as jnp
from jax import lax
from jax.experimental import pallas as pl
from jax.experimental.pallas import tpu as pltpu
from jax.experimental.pallas import tpu_sc as plsc


def _mlp_body(table_ref, w1_ref, b1_ref, w2_ref, b2_ref, out_ref):
    x = table_ref[...]
    h = jnp.dot(x, w1_ref[...], preferred_element_type=jnp.float32) + b1_ref[...]
    h = h * (1.0 / (1.0 + jnp.exp(-h)))
    y = jnp.dot(h, w2_ref[...], preferred_element_type=jnp.float32) + b2_ref[...]
    out_ref[...] = y * (1.0 / (1.0 + jnp.exp(-y)))


def _transform_table(table, W1, b1, W2, b2):
    V = table.shape[0]
    P = W2.shape[1]
    return pl.pallas_call(
        _mlp_body,
        out_shape=jax.ShapeDtypeStruct((V, P), jnp.float32),
    )(table, W1, b1.reshape(1, -1), W2, b2.reshape(1, -1))


@functools.lru_cache(maxsize=None)
def _make_gather(V, D, B, nchunks=4):
    info = plsc.get_sparse_core_info()
    nc, ns = info.num_cores, info.num_subcores
    nw = nc * ns
    b_per_w = B // nw
    cs = b_per_w // nchunks
    mesh = plsc.VectorSubcoreMesh(core_axis_name="c", subcore_axis_name="s")

    @functools.partial(
        pl.kernel,
        mesh=mesh,
        out_type=jax.ShapeDtypeStruct((B, D), jnp.float32),
        scratch_types=[
            pltpu.VMEM((b_per_w,), jnp.int32),
            pltpu.VMEM((cs, D), jnp.float32),
            pltpu.VMEM((cs, D), jnp.float32),
            pltpu.SemaphoreType.DMA,
            pltpu.SemaphoreType.DMA,
            pltpu.SemaphoreType.DMA,
            pltpu.SemaphoreType.DMA,
        ],
    )
    def gather(idx_hbm, table_hbm, out_hbm, idx_v, rows0, rows1, g0, g1, w0, w1):
        wid = lax.axis_index("s") * nc + lax.axis_index("c")
        base = wid * b_per_w
        pltpu.sync_copy(idx_hbm.at[pl.ds(base, b_per_w)], idx_v)
        bufs = (rows0, rows1)
        gsems = (g0, g1)
        wsems = (w0, w1)
        gathers = [None] * nchunks
        writes = [None] * nchunks
        gathers[0] = pltpu.async_copy(
            table_hbm.at[idx_v.at[pl.ds(0, cs)]], bufs[0], gsems[0])
        for c in range(nchunks):
            if c + 1 < nchunks:
                b = (c + 1) % 2
                if writes[c - 1] is not None:
                    writes[c - 1].wait()
                    writes[c - 1] = None
                gathers[c + 1] = pltpu.async_copy(
                    table_hbm.at[idx_v.at[pl.ds((c + 1) * cs, cs)]],
                    bufs[b], gsems[b])
            gathers[c].wait()
            writes[c] = pltpu.async_copy(
                bufs[c % 2], out_hbm.at[pl.ds(base + c * cs, cs)],
                wsems[c % 2])
        for c in (nchunks - 2, nchunks - 1):
            if writes[c] is not None:
                writes[c].wait()

    return gather


def kernel(t, table, W1, b1, W2, b2):
    ytab = _transform_table(table, W1, b1, W2, b2)
    gather = _make_gather(table.shape[0], table.shape[1], t.shape[0])
    return gather(t, ytab)


# speedup vs baseline: 1.9318x; 1.9318x over previous
import functools

import jax
import jax.numpy as jnp
from jax import lax
from jax.experimental import pallas as pl
from jax.experimental.pallas import tpu as pltpu
from jax.experimental.pallas import tpu_sc as plsc

_BLOCK = 2048


def _fused_body(t_ref, table_ref, w1_ref, b1_ref, w2_ref, b2_ref,
                out_ref, ytab_ref):
    @pl.when(pl.program_id(0) == 0)
    def _():
        x = table_ref[...]
        h = jnp.dot(x, w1_ref[...], preferred_element_type=jnp.float32) + b1_ref[...]
        h = h * (1.0 / (1.0 + jnp.exp(-h)))
        y = jnp.dot(h, w2_ref[...], preferred_element_type=jnp.float32) + b2_ref[...]
        y = y * (1.0 / (1.0 + jnp.exp(-y)))
        ytab_ref[...] = y.astype(jnp.bfloat16)

    idx = t_ref[0, 0, :]
    vpad = ytab_ref.shape[0]
    block = out_ref.shape[0]
    iota = lax.broadcasted_iota(jnp.int32, (block, vpad), 1)
    onehot = jnp.where(idx[:, None] == iota, 1.0, 0.0).astype(jnp.bfloat16)
    out_ref[...] = jnp.dot(onehot, ytab_ref[...],
                           preferred_element_type=jnp.float32)


def kernel(t, table, W1, b1, W2, b2):
    V, D = table.shape
    P = W2.shape[1]
    B = t.shape[0]
    vpad = (V + 127) // 128 * 128
    table_p = jnp.pad(table, ((0, vpad - V), (0, 0)))
    nb = B // _BLOCK
    t3 = t.reshape(nb, 1, _BLOCK)
    return pl.pallas_call(
        _fused_body,
        grid=(nb,),
        in_specs=[
            pl.BlockSpec((1, 1, _BLOCK), lambda b: (b, 0, 0)),
            pl.BlockSpec((vpad, D), lambda b: (0, 0)),
            pl.BlockSpec((D, P), lambda b: (0, 0)),
            pl.BlockSpec((1, P), lambda b: (0, 0)),
            pl.BlockSpec((P, P), lambda b: (0, 0)),
            pl.BlockSpec((1, P), lambda b: (0, 0)),
        ],
        out_specs=pl.BlockSpec((_BLOCK, P), lambda b: (b, 0)),
        out_shape=jax.ShapeDtypeStruct((B, P), jnp.float32),
        scratch_shapes=[pltpu.VMEM((vpad, P), jnp.bfloat16)],
    )(t3, table_p, W1, b1.reshape(1, -1), W2, b2.reshape(1, -1))


# speedup vs baseline: 1.9891x; 1.0297x over previous
import functools

import jax
import jax.numpy as jnp
from jax import lax
from jax.experimental import pallas as pl
from jax.experimental.pallas import tpu as pltpu
from jax.experimental.pallas import tpu_sc as plsc

_BLOCK = 8192


def _fused_body(t_ref, table_ref, w1_ref, b1_ref, w2_ref, b2_ref,
                out_ref, ytab_ref):
    @pl.when(pl.program_id(0) == 0)
    def _():
        x = table_ref[...]
        h = jnp.dot(x, w1_ref[...], preferred_element_type=jnp.float32) + b1_ref[...]
        h = h * (1.0 / (1.0 + jnp.exp(-h)))
        y = jnp.dot(h, w2_ref[...], preferred_element_type=jnp.float32) + b2_ref[...]
        y = y * (1.0 / (1.0 + jnp.exp(-y)))
        ytab_ref[...] = y.astype(jnp.bfloat16)

    idx = t_ref[0, 0, :]
    vpad = ytab_ref.shape[0]
    block = out_ref.shape[0]
    iota = lax.broadcasted_iota(jnp.int16, (block, vpad), 1)
    cond = idx.astype(jnp.int16)[:, None] == iota
    onehot = jnp.where(cond, jnp.bfloat16(1), jnp.bfloat16(0))
    out_ref[...] = jnp.dot(onehot, ytab_ref[...],
                           preferred_element_type=jnp.float32)


def kernel(t, table, W1, b1, W2, b2):
    V, D = table.shape
    P = W2.shape[1]
    B = t.shape[0]
    vpad = (V + 127) // 128 * 128
    table_p = jnp.pad(table, ((0, vpad - V), (0, 0)))
    nb = B // _BLOCK
    t3 = t.reshape(nb, 1, _BLOCK)
    return pl.pallas_call(
        _fused_body,
        grid=(nb,),
        in_specs=[
            pl.BlockSpec((1, 1, _BLOCK), lambda b: (b, 0, 0)),
            pl.BlockSpec((vpad, D), lambda b: (0, 0)),
            pl.BlockSpec((D, P), lambda b: (0, 0)),
            pl.BlockSpec((1, P), lambda b: (0, 0)),
            pl.BlockSpec((P, P), lambda b: (0, 0)),
            pl.BlockSpec((1, P), lambda b: (0, 0)),
        ],
        out_specs=pl.BlockSpec((_BLOCK, P), lambda b: (b, 0)),
        out_shape=jax.ShapeDtypeStruct((B, P), jnp.float32),
        scratch_shapes=[pltpu.VMEM((vpad, P), jnp.bfloat16)],
    )(t3, table_p, W1, b1.reshape(1, -1), W2, b2.reshape(1, -1))


# speedup vs baseline: 2.0167x; 1.0139x over previous
import functools

import jax
import jax.numpy as jnp
from jax import lax
from jax.experimental import pallas as pl
from jax.experimental.pallas import tpu as pltpu
from jax.experimental.pallas import tpu_sc as plsc

_BLOCK = 4096


def _fused_body(t_ref, table_ref, w1_ref, b1_ref, w2_ref, b2_ref,
                out_ref, ytab_ref):
    @pl.when(pl.program_id(0) == 0)
    def _():
        x = table_ref[...]
        h = jnp.dot(x, w1_ref[...], preferred_element_type=jnp.float32) + b1_ref[...]
        h = h * (1.0 / (1.0 + jnp.exp(-h)))
        y = jnp.dot(h, w2_ref[...], preferred_element_type=jnp.float32) + b2_ref[...]
        y = y * (1.0 / (1.0 + jnp.exp(-y)))
        ytab_ref[...] = y.astype(jnp.bfloat16)

    idx = t_ref[0, 0, :]
    vpad = ytab_ref.shape[0]
    block = out_ref.shape[0]
    iota = lax.broadcasted_iota(jnp.int16, (block, vpad), 1)
    cond = idx.astype(jnp.int16)[:, None] == iota
    onehot = jnp.where(cond, jnp.bfloat16(1), jnp.bfloat16(0))
    out_ref[...] = jnp.dot(onehot, ytab_ref[...],
                           preferred_element_type=jnp.float32)


def kernel(t, table, W1, b1, W2, b2):
    V, D = table.shape
    P = W2.shape[1]
    B = t.shape[0]
    vpad = (V + 127) // 128 * 128
    table_p = jnp.pad(table, ((0, vpad - V), (0, 0)))
    nb = B // _BLOCK
    t3 = t.reshape(nb, 1, _BLOCK)
    return pl.pallas_call(
        _fused_body,
        grid=(nb,),
        in_specs=[
            pl.BlockSpec((1, 1, _BLOCK), lambda b: (b, 0, 0)),
            pl.BlockSpec((vpad, D), lambda b: (0, 0)),
            pl.BlockSpec((D, P), lambda b: (0, 0)),
            pl.BlockSpec((1, P), lambda b: (0, 0)),
            pl.BlockSpec((P, P), lambda b: (0, 0)),
            pl.BlockSpec((1, P), lambda b: (0, 0)),
        ],
        out_specs=pl.BlockSpec((_BLOCK, P), lambda b: (b, 0)),
        out_shape=jax.ShapeDtypeStruct((B, P), jnp.float32),
        scratch_shapes=[pltpu.VMEM((vpad, P), jnp.bfloat16)],
    )(t3, table_p, W1, b1.reshape(1, -1), W2, b2.reshape(1, -1))


# speedup vs baseline: 2.3067x; 1.1438x over previous
import functools

import jax
import jax.numpy as jnp
from jax import lax
from jax.experimental import pallas as pl
from jax.experimental.pallas import tpu as pltpu
from jax.experimental.pallas import tpu_sc as plsc

_BLOCK = 4096


def _fused_body(nrows, t_ref, table_ref, w1_ref, b1_ref, w2_ref, b2_ref,
                out_ref, ytab_ref):
    @pl.when(pl.program_id(0) == 0)
    def _():
        x = table_ref[...]
        h = jnp.dot(x, w1_ref[...], preferred_element_type=jnp.float32) + b1_ref[...]
        h = h * (1.0 / (1.0 + jnp.exp(-h)))
        y = jnp.dot(h, w2_ref[...], preferred_element_type=jnp.float32) + b2_ref[...]
        y = y * (1.0 / (1.0 + jnp.exp(-y)))
        row = lax.broadcasted_iota(jnp.int32, y.shape, 0)
        ytab_ref[...] = jnp.where(row < nrows, y, 0.0).astype(jnp.bfloat16)

    idx = t_ref[0, 0, :]
    vpad = ytab_ref.shape[0]
    block = out_ref.shape[0]
    iota = lax.broadcasted_iota(jnp.int16, (block, vpad), 1)
    cond = idx.astype(jnp.int16)[:, None] == iota
    onehot = jnp.where(cond, jnp.bfloat16(1), jnp.bfloat16(0))
    out_ref[...] = jnp.dot(onehot, ytab_ref[...],
                           preferred_element_type=jnp.float32)


def kernel(t, table, W1, b1, W2, b2):
    V, D = table.shape
    P = W2.shape[1]
    B = t.shape[0]
    vpad = (V + 127) // 128 * 128
    nb = B // _BLOCK
    t3 = t.reshape(nb, 1, _BLOCK)
    return pl.pallas_call(
        functools.partial(_fused_body, V),
        grid=(nb,),
        in_specs=[
            pl.BlockSpec((1, 1, _BLOCK), lambda b: (b, 0, 0)),
            pl.BlockSpec((vpad, D), lambda b: (0, 0)),
            pl.BlockSpec((D, P), lambda b: (0, 0)),
            pl.BlockSpec((1, P), lambda b: (0, 0)),
            pl.BlockSpec((P, P), lambda b: (0, 0)),
            pl.BlockSpec((1, P), lambda b: (0, 0)),
        ],
        out_specs=pl.BlockSpec((_BLOCK, P), lambda b: (b, 0)),
        out_shape=jax.ShapeDtypeStruct((B, P), jnp.float32),
        scratch_shapes=[pltpu.VMEM((vpad, P), jnp.bfloat16)],
    )(t3, table, W1, b1.reshape(1, -1), W2, b2.reshape(1, -1))
